# baseline (speedup 1.0000x reference)
import jax
import jax.numpy as jnp
from jax import lax
from jax.experimental import pallas as pl
from jax.experimental.pallas import tpu as pltpu
from jax.experimental.pallas import tpu_sc as plsc

D = 64
LN_EPS = 1e-5
NC = 2
NS = 16
NW = NC * NS

_GATHER_DNUMS = lax.GatherDimensionNumbers(
    offset_dims=(), collapsed_slice_dims=(0,), start_index_map=(0,))


def _lane_sum(x):
    lane = lax.iota(jnp.int32, 16)
    for s in (1, 2, 4, 8):
        p = (lane ^ s).reshape(16, 1)
        x = x + lax.gather(x, p, _GATHER_DNUMS, (1,),
                           mode=lax.GatherScatterMode.PROMISE_IN_BOUNDS)
    return x


def _ln_row_t(gbuf, tbuf, r, g_regs, b_regs, kbase):
    v = [gbuf[r, pl.ds(16 * k, 16)] for k in range(4)]
    s = _lane_sum(v[0] + v[1] + v[2] + v[3])
    ss = _lane_sum(v[0] * v[0] + (v[1] * v[1] + (v[2] * v[2] + v[3] * v[3])))
    mean = s * (1.0 / 64.0)
    var = ss * (1.0 / 64.0) - mean * mean
    x = var + LN_EPS
    i = lax.bitcast_convert_type(x, jnp.int32)
    i = jnp.int32(0x5F3759DF) - lax.shift_right_logical(i, 1)
    y = lax.bitcast_convert_type(i, jnp.float32)
    xh = 0.5 * x
    y = y * (1.5 - xh * y * y)
    y = y * (1.5 - xh * y * y)
    nb = -mean * y
    rv = jnp.full((16,), r, jnp.int32)
    for k in range(4):
        o = (v[k] * y + nb) * g_regs[k] + b_regs[k]
        plsc.store_scatter(tbuf, [kbase[k], rv], o)


def _make_sc_call(batch, n_l, chunk):
    stripe = batch // NW
    halves = stripe // chunk
    n_chunks = n_l * halves
    n_out_rows = n_l * D * (batch // chunk)
    mesh = plsc.VectorSubcoreMesh(core_axis_name="c", subcore_axis_name="s")

    def body(x_hbm, tab_hbm, g_hbm, b_hbm, out_hbm,
             idx_all, g0, g1, t0, t1, sidx, g_v, b_v,
             sg0, sg1, so0, so1):
        wid = lax.axis_index("s") * NC + lax.axis_index("c")
        pltpu.sync_copy(x_hbm.at[wid], idx_all)
        pltpu.sync_copy(g_hbm, g_v)
        pltpu.sync_copy(b_hbm, b_v)
        g_regs = [g_v[pl.ds(16 * k, 16)] for k in range(4)]
        b_regs = [b_v[pl.ds(16 * k, 16)] for k in range(4)]
        lane = lax.iota(jnp.int32, 16)
        kbase = [lane + (16 * k) for k in range(4)]
        bufs = ((g0, t0, sg0, so0), (g1, t1, sg1, so1))

        def fill_sidx(c, p):
            col = wid * halves + (c & (halves - 1))
            base = (c >> 1) * (D * 64) + col
            for k in range(4):
                sidx[p, pl.ds(16 * k, 16)] = (kbase[k] * 64) + base

        def fire_gather(c, g, sg):
            pltpu.async_copy(
                tab_hbm.at[idx_all.at[c >> 1, c & (halves - 1)]], g, sg)

        def wait_gather(c, g, sg):
            pltpu.make_async_copy(
                tab_hbm.at[idx_all.at[c >> 1, c & (halves - 1)]], g, sg
            ).wait()

        def fire_out(p, t, so):
            pltpu.async_copy(t, out_hbm.at[sidx.at[p]], so)

        def wait_out(p, t, so):
            pltpu.make_async_copy(t, out_hbm.at[sidx.at[p]], so).wait()

        def compute(g, t):
            @plsc.parallel_loop(0, chunk, unroll=4)
            def _row(r):
                _ln_row_t(g, t, r, g_regs, b_regs, kbase)

        def step(c, p, first):
            g_m, t_m, sg_m, so_m = bufs[p]
            g_o, t_o, sg_o, so_o = bufs[1 - p]
            if not first:
                wait_out(1 - p, t_o, so_o)

            @pl.when(c + 1 < n_chunks)
            def _():
                fire_gather(c + 1, g_o, sg_o)

            wait_gather(c, g_m, sg_m)
            compute(g_m, t_m)
            fill_sidx(c, p)
            fire_out(p, t_m, so_m)

        fire_gather(0, g0, sg0)
        step(0, 0, first=True)

        @pl.loop(1, n_chunks)
        def _chunk(c):
            @pl.when((c & 1) == 1)
            def _():
                step(c, 1, first=False)

            @pl.when((c & 1) == 0)
            def _():
                step(c, 0, first=False)

        last = (n_chunks - 1) & 1
        wait_out(last, bufs[last][1], bufs[last][3])

    return pl.kernel(
        body,
        out_type=jax.ShapeDtypeStruct((n_out_rows, chunk), jnp.float32),
        mesh=mesh,
        scratch_types=[
            pltpu.VMEM((n_l, halves, chunk), jnp.int32),
            pltpu.VMEM((chunk, D), jnp.float32),
            pltpu.VMEM((chunk, D), jnp.float32),
            pltpu.VMEM((D, chunk), jnp.float32),
            pltpu.VMEM((D, chunk), jnp.float32),
            pltpu.VMEM((2, D), jnp.int32),
            pltpu.VMEM((D,), jnp.float32),
            pltpu.VMEM((D,), jnp.float32),
            pltpu.SemaphoreType.DMA,
            pltpu.SemaphoreType.DMA,
            pltpu.SemaphoreType.DMA,
            pltpu.SemaphoreType.DMA,
        ],
        compiler_params=pltpu.CompilerParams(
            use_tc_tiling_on_sc=False, needs_layout_passes=False),
    )


def kernel(x, table, gamma, beta):
    b, l = x.shape
    chunk = 256
    xt = x.T.reshape(l, NW, b // (NW * chunk), chunk).transpose(1, 0, 2, 3)
    out = _make_sc_call(b, l, chunk)(xt, table, gamma, beta)
    return out.reshape(l, D, b).transpose(2, 0, 1)


# speedup vs baseline: 1.5075x; 1.5075x over previous
import jax
import jax.numpy as jnp
from jax import lax
from jax.experimental import pallas as pl
from jax.experimental.pallas import tpu as pltpu
from jax.experimental.pallas import tpu_sc as plsc

D = 64
LN_EPS = 1e-5
NC = 2
NS = 16
NW = NC * NS
TPAD = 273

_GATHER_DNUMS = lax.GatherDimensionNumbers(
    offset_dims=(), collapsed_slice_dims=(0,), start_index_map=(0,))


def _lane_sum(x):
    lane = lax.iota(jnp.int32, 16)
    for s in (1, 2, 4, 8):
        p = (lane ^ s).reshape(16, 1)
        x = x + lax.gather(x, p, _GATHER_DNUMS, (1,),
                           mode=lax.GatherScatterMode.PROMISE_IN_BOUNDS)
    return x


def _ln_row_t(gbuf, tbuf, r, g_regs, b_regs, kbase):
    v = [gbuf[r, pl.ds(16 * k, 16)] for k in range(4)]
    s = _lane_sum(v[0] + v[1] + v[2] + v[3])
    ss = _lane_sum(v[0] * v[0] + (v[1] * v[1] + (v[2] * v[2] + v[3] * v[3])))
    mean = s * (1.0 / 64.0)
    var = ss * (1.0 / 64.0) - mean * mean
    x = var + LN_EPS
    i = lax.bitcast_convert_type(x, jnp.int32)
    i = jnp.int32(0x5F3759DF) - lax.shift_right_logical(i, 1)
    y = lax.bitcast_convert_type(i, jnp.float32)
    xh = 0.5 * x
    y = y * (1.5 - xh * y * y)
    y = y * (1.5 - xh * y * y)
    nb = -mean * y
    rv = jnp.full((16,), r, jnp.int32)
    for k in range(4):
        o = (v[k] * y + nb) * g_regs[k] + b_regs[k]
        plsc.store_scatter(tbuf, [kbase[k], rv], o)


def _make_sc_call(batch, n_l, chunk):
    stripe = batch // NW
    halves = stripe // chunk
    n_chunks = n_l * halves
    n_cols = batch // chunk
    mesh = plsc.VectorSubcoreMesh(core_axis_name="c", subcore_axis_name="s")

    def body(x_hbm, tab_hbm, g_hbm, b_hbm, out_hbm,
             idx_all, g0, g1, t0, t1, g_v, b_v,
             sg0, sg1, so0, so1):
        wid = lax.axis_index("s") * NC + lax.axis_index("c")
        pltpu.sync_copy(x_hbm.at[wid], idx_all)
        pltpu.sync_copy(g_hbm, g_v)
        pltpu.sync_copy(b_hbm, b_v)
        g_regs = [g_v[pl.ds(16 * k, 16)] for k in range(4)]
        b_regs = [b_v[pl.ds(16 * k, 16)] for k in range(4)]
        lane = lax.iota(jnp.int32, 16)
        kbase = [lane + (16 * k) for k in range(4)]
        bufs = ((g0, t0, sg0, so0), (g1, t1, sg1, so1))

        def fire_gather(c, g, sg):
            pltpu.async_copy(
                tab_hbm.at[idx_all.at[c >> 1, c & (halves - 1)]], g, sg)

        def wait_gather(c, g, sg):
            pltpu.make_async_copy(
                tab_hbm.at[idx_all.at[c >> 1, c & (halves - 1)]], g, sg
            ).wait()

        def out_slice(c):
            col = wid * halves + (c & (halves - 1))
            return out_hbm.at[c >> 1, :, col, :]

        def fire_out(c, t, so):
            pltpu.async_copy(t.at[:, pl.ds(0, chunk)], out_slice(c), so)

        def wait_out(c, t, so):
            pltpu.make_async_copy(
                t.at[:, pl.ds(0, chunk)], out_slice(c), so).wait()

        def compute(g, t):
            @plsc.parallel_loop(0, chunk, unroll=4)
            def _row(r):
                _ln_row_t(g, t, r, g_regs, b_regs, kbase)

        def step(c, p, first):
            g_m, t_m, sg_m, so_m = bufs[p]
            g_o, t_o, sg_o, so_o = bufs[1 - p]
            if not first:
                wait_out(c - 1, t_o, so_o)

            @pl.when(c + 1 < n_chunks)
            def _():
                fire_gather(c + 1, g_o, sg_o)

            wait_gather(c, g_m, sg_m)
            compute(g_m, t_m)
            fire_out(c, t_m, so_m)

        fire_gather(0, g0, sg0)
        step(0, 0, first=True)

        @pl.loop(1, n_chunks)
        def _chunk(c):
            @pl.when((c & 1) == 1)
            def _():
                step(c, 1, first=False)

            @pl.when((c & 1) == 0)
            def _():
                step(c, 0, first=False)

        last = (n_chunks - 1) & 1
        wait_out(n_chunks - 1, bufs[last][1], bufs[last][3])

    return pl.kernel(
        body,
        out_type=jax.ShapeDtypeStruct((n_l, D, n_cols, chunk), jnp.float32),
        mesh=mesh,
        scratch_types=[
            pltpu.VMEM((n_l, halves, chunk), jnp.int32),
            pltpu.VMEM((chunk, D), jnp.float32),
            pltpu.VMEM((chunk, D), jnp.float32),
            pltpu.VMEM((D, TPAD), jnp.float32),
            pltpu.VMEM((D, TPAD), jnp.float32),
            pltpu.VMEM((D,), jnp.float32),
            pltpu.VMEM((D,), jnp.float32),
            pltpu.SemaphoreType.DMA,
            pltpu.SemaphoreType.DMA,
            pltpu.SemaphoreType.DMA,
            pltpu.SemaphoreType.DMA,
        ],
        compiler_params=pltpu.CompilerParams(
            use_tc_tiling_on_sc=False, needs_layout_passes=False),
    )


def kernel(x, table, gamma, beta):
    b, l = x.shape
    chunk = 256
    xt = x.T.reshape(l, NW, b // (NW * chunk), chunk).transpose(1, 0, 2, 3)
    out = _make_sc_call(b, l, chunk)(xt, table, gamma, beta)
    return out.reshape(l, D, b).transpose(2, 0, 1)
